# baseline (speedup 1.0000x reference)
import jax
import jax.numpy as jnp
from jax.experimental import pallas as pl
from jax.experimental.pallas import tpu as pltpu

_WIN = 24
_FEATS = 4
_N = _WIN * _FEATS
_LANE_TILE = 2048


def _gen_kernel(z_ref, w_ref, bt_ref, o_ref):
    ht = jax.lax.dot_general(
        w_ref[...], z_ref[...].astype(jnp.bfloat16),
        (((0,), (1,)), ((), ())),
        preferred_element_type=jnp.float32,
    )
    ht = (ht + bt_ref[...]) * 0.5
    o_ref[...] = jnp.tanh(ht) * 0.5 + 0.5


def kernel(z, W_bd, b_t):
    B, latent = z.shape

    bp = -(-B // _LANE_TILE) * _LANE_TILE
    if bp != B:
        z = jnp.pad(z, ((0, bp - B), (0, 0)))

    w = W_bd[:latent, :_N].astype(jnp.bfloat16)
    bt = b_t[:, :_N].reshape(_N, 1)

    ht = pl.pallas_call(
        _gen_kernel,
        out_shape=jax.ShapeDtypeStruct((_N, bp), jnp.float32),
        grid=(bp // _LANE_TILE,),
        in_specs=[
            pl.BlockSpec((_LANE_TILE, latent), lambda i: (i, 0)),
            pl.BlockSpec((latent, _N), lambda i: (0, 0)),
            pl.BlockSpec((_N, 1), lambda i: (0, 0)),
        ],
        out_specs=pl.BlockSpec((_N, _LANE_TILE), lambda i: (0, i)),
        compiler_params=pltpu.CompilerParams(
            dimension_semantics=("parallel",)),
    )(z, w, bt)

    if bp != B:
        ht = ht[:, :B]
    return ht.reshape(_WIN, _FEATS, B).transpose(2, 0, 1)


# speedup vs baseline: 1.5302x; 1.5302x over previous
import jax
import jax.numpy as jnp
from jax.experimental import pallas as pl
from jax.experimental.pallas import tpu as pltpu

_WIN = 24
_FEATS = 4
_N = _WIN * _FEATS
_LANE_TILE = 2048


def _gen_kernel(z_ref, w_ref, bt_ref, o_ref):
    ht = jax.lax.dot_general(
        w_ref[...], z_ref[...].astype(jnp.bfloat16),
        (((0,), (1,)), ((), ())),
        preferred_element_type=jnp.float32,
    )
    ht = (ht + bt_ref[...]) * 0.5
    s = jnp.tanh(ht) * 0.5 + 0.5
    o_ref[...] = s.reshape(_WIN, _FEATS, _LANE_TILE)


def kernel(z, W_bd, b_t):
    B, latent = z.shape

    bp = -(-B // _LANE_TILE) * _LANE_TILE
    if bp != B:
        z = jnp.pad(z, ((0, bp - B), (0, 0)))

    w = W_bd[:latent, :_N].astype(jnp.bfloat16)
    bt = b_t[:, :_N].reshape(_N, 1)

    y = pl.pallas_call(
        _gen_kernel,
        out_shape=jax.ShapeDtypeStruct((_WIN, _FEATS, bp), jnp.float32),
        grid=(bp // _LANE_TILE,),
        in_specs=[
            pl.BlockSpec((_LANE_TILE, latent), lambda i: (i, 0)),
            pl.BlockSpec((latent, _N), lambda i: (0, 0)),
            pl.BlockSpec((_N, 1), lambda i: (0, 0)),
        ],
        out_specs=pl.BlockSpec((_WIN, _FEATS, _LANE_TILE), lambda i: (0, 0, i)),
        compiler_params=pltpu.CompilerParams(
            dimension_semantics=("parallel",)),
    )(z, w, bt)

    if bp != B:
        y = y[:, :, :B]
    return y.transpose(2, 0, 1)


# speedup vs baseline: 2.0927x; 1.3676x over previous
import jax
import jax.numpy as jnp
from jax.experimental import pallas as pl
from jax.experimental.pallas import tpu as pltpu

_WIN = 24
_FEATS = 4
_N = _WIN * _FEATS
_LANE_TILE = 4096


def _gen_kernel(z_ref, w_ref, bt_ref, o_ref):
    ht = jax.lax.dot_general(
        w_ref[...], z_ref[...].astype(jnp.bfloat16),
        (((0,), (1,)), ((), ())),
        preferred_element_type=jnp.float32,
    )
    ht = (ht + bt_ref[...]) * 0.5
    s = jnp.tanh(ht) * 0.5 + 0.5
    o_ref[...] = s.reshape(_WIN, _FEATS, _LANE_TILE)


def kernel(z, W_bd, b_t):
    B, latent = z.shape

    bp = -(-B // _LANE_TILE) * _LANE_TILE
    if bp != B:
        z = jnp.pad(z, ((0, bp - B), (0, 0)))

    w = W_bd[:latent, :_N].astype(jnp.bfloat16)
    bt = b_t[:, :_N].reshape(_N, 1)

    y = pl.pallas_call(
        _gen_kernel,
        out_shape=jax.ShapeDtypeStruct((_WIN, _FEATS, bp), jnp.float32),
        grid=(bp // _LANE_TILE,),
        in_specs=[
            pl.BlockSpec((_LANE_TILE, latent), lambda i: (i, 0)),
            pl.BlockSpec((latent, _N), lambda i: (0, 0)),
            pl.BlockSpec((_N, 1), lambda i: (0, 0)),
        ],
        out_specs=pl.BlockSpec((_WIN, _FEATS, _LANE_TILE), lambda i: (0, 0, i)),
        compiler_params=pltpu.CompilerParams(
            dimension_semantics=("parallel",)),
    )(z, w, bt)

    if bp != B:
        y = y[:, :, :B]
    return y.transpose(2, 0, 1)


# speedup vs baseline: 2.3881x; 1.1412x over previous
import jax
import jax.numpy as jnp
from jax.experimental import pallas as pl
from jax.experimental.pallas import tpu as pltpu

_WIN = 24
_FEATS = 4
_N = _WIN * _FEATS
_LANE_TILE = 8192


def _gen_kernel(z_ref, w_ref, bt_ref, o_ref):
    ht = jax.lax.dot_general(
        w_ref[...], z_ref[...].astype(jnp.bfloat16),
        (((0,), (1,)), ((), ())),
        preferred_element_type=jnp.float32,
    )
    ht = (ht + bt_ref[...]) * 0.5
    s = jnp.tanh(ht) * 0.5 + 0.5
    o_ref[...] = s.reshape(_WIN, _FEATS, _LANE_TILE)


def kernel(z, W_bd, b_t):
    B, latent = z.shape

    bp = -(-B // _LANE_TILE) * _LANE_TILE
    if bp != B:
        z = jnp.pad(z, ((0, bp - B), (0, 0)))

    w = W_bd[:latent, :_N].astype(jnp.bfloat16)
    bt = b_t[:, :_N].reshape(_N, 1)

    y = pl.pallas_call(
        _gen_kernel,
        out_shape=jax.ShapeDtypeStruct((_WIN, _FEATS, bp), jnp.float32),
        grid=(bp // _LANE_TILE,),
        in_specs=[
            pl.BlockSpec((_LANE_TILE, latent), lambda i: (i, 0)),
            pl.BlockSpec((latent, _N), lambda i: (0, 0)),
            pl.BlockSpec((_N, 1), lambda i: (0, 0)),
        ],
        out_specs=pl.BlockSpec((_WIN, _FEATS, _LANE_TILE), lambda i: (0, 0, i)),
        compiler_params=pltpu.CompilerParams(
            dimension_semantics=("parallel",)),
    )(z, w, bt)

    if bp != B:
        y = y[:, :, :B]
    return y.transpose(2, 0, 1)


# speedup vs baseline: 2.5248x; 1.0572x over previous
import jax
import jax.numpy as jnp
from jax.experimental import pallas as pl
from jax.experimental.pallas import tpu as pltpu

_WIN = 24
_FEATS = 4
_N = _WIN * _FEATS
_LANE_TILE = 16384


def _gen_kernel(z_ref, w_ref, bt_ref, o_ref):
    ht = jax.lax.dot_general(
        w_ref[...], z_ref[...].astype(jnp.bfloat16),
        (((0,), (1,)), ((), ())),
        preferred_element_type=jnp.float32,
    )
    ht = (ht + bt_ref[...]) * 0.5
    s = jnp.tanh(ht) * 0.5 + 0.5
    o_ref[...] = s.reshape(_WIN, _FEATS, _LANE_TILE)


def kernel(z, W_bd, b_t):
    B, latent = z.shape

    bp = -(-B // _LANE_TILE) * _LANE_TILE
    if bp != B:
        z = jnp.pad(z, ((0, bp - B), (0, 0)))

    w = W_bd[:latent, :_N].astype(jnp.bfloat16)
    bt = b_t[:, :_N].reshape(_N, 1)

    y = pl.pallas_call(
        _gen_kernel,
        out_shape=jax.ShapeDtypeStruct((_WIN, _FEATS, bp), jnp.float32),
        grid=(bp // _LANE_TILE,),
        in_specs=[
            pl.BlockSpec((_LANE_TILE, latent), lambda i: (i, 0)),
            pl.BlockSpec((latent, _N), lambda i: (0, 0)),
            pl.BlockSpec((_N, 1), lambda i: (0, 0)),
        ],
        out_specs=pl.BlockSpec((_WIN, _FEATS, _LANE_TILE), lambda i: (0, 0, i)),
        compiler_params=pltpu.CompilerParams(
            dimension_semantics=("parallel",)),
    )(z, w, bt)

    if bp != B:
        y = y[:, :, :B]
    return y.transpose(2, 0, 1)


# speedup vs baseline: 2.7126x; 1.0744x over previous
import jax
import jax.numpy as jnp
from jax.experimental import pallas as pl
from jax.experimental.pallas import tpu as pltpu

_WIN = 24
_FEATS = 4
_N = _WIN * _FEATS
_LANE_TILE = 32768


def _gen_kernel(z_ref, w_ref, bt_ref, o_ref):
    ht = jax.lax.dot_general(
        w_ref[...], z_ref[...].astype(jnp.bfloat16),
        (((0,), (1,)), ((), ())),
        preferred_element_type=jnp.float32,
    )
    ht = (ht + bt_ref[...]) * 0.5
    s = jnp.tanh(ht) * 0.5 + 0.5
    o_ref[...] = s.reshape(_WIN, _FEATS, _LANE_TILE)


def kernel(z, W_bd, b_t):
    B, latent = z.shape

    bp = -(-B // _LANE_TILE) * _LANE_TILE
    if bp != B:
        z = jnp.pad(z, ((0, bp - B), (0, 0)))

    w = W_bd[:latent, :_N].astype(jnp.bfloat16)
    bt = b_t[:, :_N].reshape(_N, 1)

    y = pl.pallas_call(
        _gen_kernel,
        out_shape=jax.ShapeDtypeStruct((_WIN, _FEATS, bp), jnp.float32),
        grid=(bp // _LANE_TILE,),
        in_specs=[
            pl.BlockSpec((_LANE_TILE, latent), lambda i: (i, 0)),
            pl.BlockSpec((latent, _N), lambda i: (0, 0)),
            pl.BlockSpec((_N, 1), lambda i: (0, 0)),
        ],
        out_specs=pl.BlockSpec((_WIN, _FEATS, _LANE_TILE), lambda i: (0, 0, i)),
        compiler_params=pltpu.CompilerParams(
            dimension_semantics=("parallel",)),
    )(z, w, bt)

    if bp != B:
        y = y[:, :, :B]
    return y.transpose(2, 0, 1)


# speedup vs baseline: 3.0764x; 1.1341x over previous
import jax
import jax.numpy as jnp
from jax.experimental import pallas as pl
from jax.experimental.pallas import tpu as pltpu

_WIN = 24
_FEATS = 4
_N = _WIN * _FEATS
_LANE_TILE = 32768


def _gen_kernel(z_ref, w_ref, b_ref, o_ref):
    w = w_ref[...][:, :_N].astype(jnp.bfloat16)
    bt = jnp.transpose(b_ref[...][:, :_N], (1, 0))
    ht = jax.lax.dot_general(
        w, z_ref[...].astype(jnp.bfloat16),
        (((0,), (1,)), ((), ())),
        preferred_element_type=jnp.float32,
    )
    ht = (ht + bt) * 0.5
    s = jnp.tanh(ht) * 0.5 + 0.5
    o_ref[...] = s.reshape(_WIN, _FEATS, _LANE_TILE)


def kernel(z, W_bd, b_t):
    B, latent = z.shape

    bp = -(-B // _LANE_TILE) * _LANE_TILE
    if bp != B:
        z = jnp.pad(z, ((0, bp - B), (0, 0)))

    y = pl.pallas_call(
        _gen_kernel,
        out_shape=jax.ShapeDtypeStruct((_WIN, _FEATS, bp), jnp.float32),
        grid=(bp // _LANE_TILE,),
        in_specs=[
            pl.BlockSpec((_LANE_TILE, latent), lambda i: (i, 0)),
            pl.BlockSpec((latent, 4 * _N), lambda i: (0, 0)),
            pl.BlockSpec((1, 4 * _N), lambda i: (0, 0)),
        ],
        out_specs=pl.BlockSpec((_WIN, _FEATS, _LANE_TILE), lambda i: (0, 0, i)),
        compiler_params=pltpu.CompilerParams(
            dimension_semantics=("parallel",)),
    )(z, W_bd, b_t)

    if bp != B:
        y = y[:, :, :B]
    return y.transpose(2, 0, 1)
